# baseline (speedup 1.0000x reference)
import functools

import jax
import jax.numpy as jnp
from jax import lax
from jax.experimental import pallas as pl
from jax.experimental.pallas import tpu as pltpu
from jax.experimental.pallas import tpu_sc as plsc

B = 16384
D = 64
NC = 2
NS = 16
NW = NC * NS
BPW = B // NW
HBUF = 256


def _sc_gather_body(gmf_u, gmf_i, mlp_u, mlp_i, uidx, iidx,
                    gu_out, gi_out, mu_out, mi_out,
                    uidx_v, iidx_v, buf_a, buf_b, sem_a, sem_b):
    wid = lax.axis_index("s") * NC + lax.axis_index("c")
    base = wid * BPW
    pltpu.sync_copy(uidx.at[pl.ds(base, BPW)], uidx_v)
    pltpu.sync_copy(iidx.at[pl.ds(base, BPW)], iidx_v)

    def gather_pair(tab_u, tab_i, out_u, out_i):
        for h in range(BPW // HBUF):
            h0 = h * HBUF

            @pl.loop(0, HBUF // 16)
            def _chunk(c):
                k0 = h0 + c * 16
                vu = uidx_v[pl.ds(k0, 16)]
                vi = iidx_v[pl.ds(k0, 16)]
                for l in range(16):
                    pltpu.async_copy(tab_u.at[pl.ds(vu[l], 1)],
                                     buf_a.at[pl.ds(c * 16 + l, 1)], sem_a)
                    pltpu.async_copy(tab_i.at[pl.ds(vi[l], 1)],
                                     buf_b.at[pl.ds(c * 16 + l, 1)], sem_b)
            pltpu.make_async_copy(out_u.at[pl.ds(base, HBUF)], buf_a, sem_a).wait()
            pltpu.make_async_copy(out_i.at[pl.ds(base, HBUF)], buf_b, sem_b).wait()
            pltpu.sync_copy(buf_a, out_u.at[pl.ds(base + h0, HBUF)])
            pltpu.sync_copy(buf_b, out_i.at[pl.ds(base + h0, HBUF)])

    gather_pair(gmf_u, gmf_i, gu_out, gi_out)
    gather_pair(mlp_u, mlp_i, mu_out, mi_out)


def _sc_gather(gmf_user, gmf_item, mlp_user, mlp_item, uidx, iidx):
    mesh = plsc.VectorSubcoreMesh(core_axis_name="c", subcore_axis_name="s")
    run = functools.partial(
        pl.kernel,
        out_type=[jax.ShapeDtypeStruct((B, D), jnp.float32)] * 4,
        mesh=mesh,
        scratch_types=[
            pltpu.VMEM((BPW,), jnp.int32),
            pltpu.VMEM((BPW,), jnp.int32),
            pltpu.VMEM((HBUF, D), jnp.float32),
            pltpu.VMEM((HBUF, D), jnp.float32),
            pltpu.SemaphoreType.DMA,
            pltpu.SemaphoreType.DMA,
        ],
    )(_sc_gather_body)
    return run(gmf_user, gmf_item, mlp_user, mlp_item, uidx, iidx)


NU = 100000
CB = 4096


def _stage1_body(gu_t, gi_t, mu_t, mi_t, wgu, wgi, wmu, wmi,
                 gu_o, gi_o, mu_o, mi_o):
    dn = (((0,), (0,)), ((), ()))
    for x_t, w, o in ((gu_t, wgu, gu_o), (gi_t, wgi, gi_o),
                      (mu_t, wmu, mu_o), (mi_t, wmi, mi_o)):
        o[...] = jax.lax.dot_general(
            x_t[...], w[...], dimension_numbers=dn,
            preferred_element_type=jnp.float32)


def _stage1(gu_t, gi_t, mu_t, mi_t, wgu, wgi, wmu, wmi):
    n = gu_t.shape[1]
    col_spec = pl.BlockSpec((D, CB), lambda i: (0, i))
    w_spec = pl.BlockSpec((D, D), lambda i: (0, 0))
    out_spec = pl.BlockSpec((CB, D), lambda i: (i, 0))
    return pl.pallas_call(
        _stage1_body,
        grid=(pl.cdiv(n, CB),),
        in_specs=[col_spec] * 4 + [w_spec] * 4,
        out_specs=[out_spec] * 4,
        out_shape=[jax.ShapeDtypeStruct((n, D), jnp.float32)] * 4,
    )(gu_t, gi_t, mu_t, mi_t, wgu, wgi, wmu, wmi)


TILE = 2048


def _dense_body(gu, gi, mu, mi, b1, w2, b2, w3, b3, wo2, bo, out):
    h = jnp.maximum(mu[...] + mi[...] + b1[...], 0.0)
    h = jnp.maximum(
        jnp.dot(h, w2[...], preferred_element_type=jnp.float32) + b2[...], 0.0)
    h = jnp.maximum(
        jnp.dot(h, w3[...], preferred_element_type=jnp.float32) + b3[...], 0.0)
    logit = (jnp.sum(gu[...] * gi[...], axis=1, keepdims=True)
             + jnp.sum(h * wo2[...], axis=1, keepdims=True) + bo[...])
    out[...] = 1.0 / (1.0 + jnp.exp(-logit))


def _dense(gu, gi, mu, mi, b1, w2, b2, w3, b3, wo2, bo):
    row_spec = pl.BlockSpec((TILE, D), lambda i: (i, 0))
    full = lambda shape: pl.BlockSpec(shape, lambda i: (0, 0))
    return pl.pallas_call(
        _dense_body,
        grid=(B // TILE,),
        in_specs=[
            row_spec, row_spec, row_spec, row_spec,
            full((1, 64)),
            full((64, 32)), full((1, 32)),
            full((32, 16)), full((1, 16)),
            full((1, 16)), full((1, 1)),
        ],
        out_specs=pl.BlockSpec((TILE, 1), lambda i: (i, 0)),
        out_shape=jax.ShapeDtypeStruct((B, 1), jnp.float32),
    )(gu, gi, mu, mi, b1, w2, b2, w3, b3, wo2, bo)


def kernel(user_input, item_input, gmf_user, gmf_item, mlp_user, mlp_item,
           W1, b1, W2, b2, W3, b3, Wo, bo):
    uidx = user_input.astype(jnp.int32)
    iidx = item_input.astype(jnp.int32)

    ones = jnp.ones((), jnp.float32)
    diag_wo = jnp.diag(Wo[:D, 0])
    diag_one = jnp.diag(jnp.broadcast_to(ones, (D,)))
    Gu_t, Gi_t, Au_t, Ai_t = _stage1(
        gmf_user.T, gmf_item.T, mlp_user.T, mlp_item.T,
        diag_wo, diag_one, W1[:D], W1[D:])

    gu, gi, mu, mi = _sc_gather(Gu_t, Gi_t, Au_t, Ai_t, uidx, iidx)

    return _dense(
        gu, gi, mu, mi,
        b1.reshape(1, 64), W2, b2.reshape(1, 32), W3, b3.reshape(1, 16),
        Wo[D:, 0].reshape(1, 16), bo.reshape(1, 1))


# speedup vs baseline: 1.1586x; 1.1586x over previous
import functools

import jax
import jax.numpy as jnp
from jax import lax
from jax.experimental import pallas as pl
from jax.experimental.pallas import tpu as pltpu
from jax.experimental.pallas import tpu_sc as plsc

B = 16384
D = 64
D2 = 2 * D
NC = 2
NS = 16
NW = NC * NS
BPW = B // NW
HBUF = 256
CHUNK = 128
NCK = BPW // CHUNK

CB = 4096


def _stage1_body(gu_t, gi_t, mu_t, mi_t, wgu, wgi, wmu, wmi, u_o, i_o):
    dn = (((0,), (0,)), ((), ()))

    def two(a_t, wa, b_t, wb):
        a = lax.dot_general(a_t[...], wa[...], dimension_numbers=dn,
                            preferred_element_type=jnp.float32)
        b = lax.dot_general(b_t[...], wb[...], dimension_numbers=dn,
                            preferred_element_type=jnp.float32)
        return jnp.concatenate([a, b], axis=1)

    u_o[...] = two(gu_t, wgu, mu_t, wmu)
    i_o[...] = two(gi_t, wgi, mi_t, wmi)


def _stage1(gu_t, gi_t, mu_t, mi_t, wgu, wgi, wmu, wmi):
    n = gu_t.shape[1]
    col_spec = pl.BlockSpec((D, CB), lambda i: (0, i))
    w_spec = pl.BlockSpec((D, D), lambda i: (0, 0))
    out_spec = pl.BlockSpec((CB, D2), lambda i: (i, 0))
    return pl.pallas_call(
        _stage1_body,
        grid=(pl.cdiv(n, CB),),
        in_specs=[col_spec] * 4 + [w_spec] * 4,
        out_specs=[out_spec] * 2,
        out_shape=[jax.ShapeDtypeStruct((n, D2), jnp.float32)] * 2,
    )(gu_t, gi_t, mu_t, mi_t, wgu, wgi, wmu, wmi)


def _sc_gather_body(u_tab, i_tab, uidx, iidx, u_out, i_out,
                    uidx_v, iidx_v, buf_a, buf_b, sem_a, sem_b):
    wid = lax.axis_index("s") * NC + lax.axis_index("c")
    base = wid * BPW
    row = wid * NCK
    pltpu.sync_copy(uidx.at[pl.ds(row, NCK)], uidx_v)
    pltpu.sync_copy(iidx.at[pl.ds(row, NCK)], iidx_v)

    for h in range(BPW // HBUF):
        cps = []
        for j in range(HBUF // CHUNK):
            c = h * (HBUF // CHUNK) + j
            cps.append(pltpu.async_copy(
                u_tab.at[uidx_v.at[c]],
                buf_a.at[pl.ds(j * CHUNK, CHUNK)], sem_a))
            cps.append(pltpu.async_copy(
                i_tab.at[iidx_v.at[c]],
                buf_b.at[pl.ds(j * CHUNK, CHUNK)], sem_b))
        for cp in cps:
            cp.wait()
        pltpu.sync_copy(buf_a, u_out.at[pl.ds(base + h * HBUF, HBUF)])
        pltpu.sync_copy(buf_b, i_out.at[pl.ds(base + h * HBUF, HBUF)])


def _sc_gather(u_tab, i_tab, uidx, iidx):
    mesh = plsc.VectorSubcoreMesh(core_axis_name="c", subcore_axis_name="s")
    run = functools.partial(
        pl.kernel,
        out_type=[jax.ShapeDtypeStruct((B, D2), jnp.float32)] * 2,
        mesh=mesh,
        scratch_types=[
            pltpu.VMEM((NCK, CHUNK), jnp.int32),
            pltpu.VMEM((NCK, CHUNK), jnp.int32),
            pltpu.VMEM((HBUF, D2), jnp.float32),
            pltpu.VMEM((HBUF, D2), jnp.float32),
            pltpu.SemaphoreType.DMA,
            pltpu.SemaphoreType.DMA,
        ],
    )(_sc_gather_body)
    return run(u_tab, i_tab, uidx, iidx)


TILE = 2048


def _dense_body(u_r, i_r, b1, w2, b2, w3, b3, wo2, bo, out):
    gu = u_r[:, :D]
    mu = u_r[:, D:]
    gi = i_r[:, :D]
    mi = i_r[:, D:]
    h = jnp.maximum(mu + mi + b1[...], 0.0)
    h = jnp.maximum(
        jnp.dot(h, w2[...], preferred_element_type=jnp.float32) + b2[...], 0.0)
    h = jnp.maximum(
        jnp.dot(h, w3[...], preferred_element_type=jnp.float32) + b3[...], 0.0)
    logit = (jnp.sum(gu * gi, axis=1, keepdims=True)
             + jnp.sum(h * wo2[...], axis=1, keepdims=True) + bo[...])
    out[...] = 1.0 / (1.0 + jnp.exp(-logit))


def _dense(u_r, i_r, b1, w2, b2, w3, b3, wo2, bo):
    row_spec = pl.BlockSpec((TILE, D2), lambda i: (i, 0))
    full = lambda shape: pl.BlockSpec(shape, lambda i: (0, 0))
    return pl.pallas_call(
        _dense_body,
        grid=(B // TILE,),
        in_specs=[
            row_spec, row_spec,
            full((1, 64)),
            full((64, 32)), full((1, 32)),
            full((32, 16)), full((1, 16)),
            full((1, 16)), full((1, 1)),
        ],
        out_specs=pl.BlockSpec((TILE, 1), lambda i: (i, 0)),
        out_shape=jax.ShapeDtypeStruct((B, 1), jnp.float32),
    )(u_r, i_r, b1, w2, b2, w3, b3, wo2, bo)


def kernel(user_input, item_input, gmf_user, gmf_item, mlp_user, mlp_item,
           W1, b1, W2, b2, W3, b3, Wo, bo):
    uidx = user_input.astype(jnp.int32).reshape(B // CHUNK, CHUNK)
    iidx = item_input.astype(jnp.int32).reshape(B // CHUNK, CHUNK)

    ones = jnp.ones((), jnp.float32)
    diag_wo = jnp.diag(Wo[:D, 0])
    diag_one = jnp.diag(jnp.broadcast_to(ones, (D,)))
    u_tab, i_tab = _stage1(
        gmf_user.T, gmf_item.T, mlp_user.T, mlp_item.T,
        diag_wo, diag_one, W1[:D], W1[D:])

    u_rows, i_rows = _sc_gather(u_tab, i_tab, uidx, iidx)

    return _dense(
        u_rows, i_rows,
        b1.reshape(1, 64), W2, b2.reshape(1, 32), W3, b3.reshape(1, 16),
        Wo[D:, 0].reshape(1, 16), bo.reshape(1, 1))


# speedup vs baseline: 1.1689x; 1.0089x over previous
import functools

import jax
import jax.numpy as jnp
from jax import lax
from jax.experimental import pallas as pl
from jax.experimental.pallas import tpu as pltpu
from jax.experimental.pallas import tpu_sc as plsc

B = 16384
D = 64
D2 = 2 * D
NC = 2
NS = 16
NW = NC * NS
BPW = B // NW
HBUF = 256
CHUNK = 128
NCK = BPW // CHUNK

CB = 8192


def _stage1_body(gu_t, gi_t, mu_t, mi_t, wgu, wgi, wmu, wmi, u_o, i_o):
    dn = (((0,), (0,)), ((), ()))

    def two(a_t, wa, b_t, wb):
        a = lax.dot_general(a_t[...], wa[...], dimension_numbers=dn,
                            preferred_element_type=jnp.float32)
        b = lax.dot_general(b_t[...], wb[...], dimension_numbers=dn,
                            preferred_element_type=jnp.float32)
        return jnp.concatenate([a, b], axis=1)

    u_o[...] = two(gu_t, wgu, mu_t, wmu)
    i_o[...] = two(gi_t, wgi, mi_t, wmi)


def _stage1(gu_t, gi_t, mu_t, mi_t, wgu, wgi, wmu, wmi):
    n = gu_t.shape[1]
    col_spec = pl.BlockSpec((D, CB), lambda i: (0, i))
    w_spec = pl.BlockSpec((D, D), lambda i: (0, 0))
    out_spec = pl.BlockSpec((CB, D2), lambda i: (i, 0))
    return pl.pallas_call(
        _stage1_body,
        grid=(pl.cdiv(n, CB),),
        in_specs=[col_spec] * 4 + [w_spec] * 4,
        out_specs=[out_spec] * 2,
        out_shape=[jax.ShapeDtypeStruct((n, D2), jnp.float32)] * 2,
    )(gu_t, gi_t, mu_t, mi_t, wgu, wgi, wmu, wmi)


def _sc_gather_body(u_tab, i_tab, uidx, iidx, u_out, i_out,
                    uidx_v, iidx_v, buf_a, buf_b, sem_a, sem_b):
    wid = lax.axis_index("s") * NC + lax.axis_index("c")
    base = wid * BPW
    row = wid * NCK
    pltpu.sync_copy(uidx.at[pl.ds(row, NCK)], uidx_v)
    pltpu.sync_copy(iidx.at[pl.ds(row, NCK)], iidx_v)

    for h in range(BPW // HBUF):
        cps = []
        for j in range(HBUF // CHUNK):
            c = h * (HBUF // CHUNK) + j
            cps.append(pltpu.async_copy(
                u_tab.at[uidx_v.at[c]],
                buf_a.at[pl.ds(j * CHUNK, CHUNK)], sem_a))
            cps.append(pltpu.async_copy(
                i_tab.at[iidx_v.at[c]],
                buf_b.at[pl.ds(j * CHUNK, CHUNK)], sem_b))
        for cp in cps:
            cp.wait()
        pltpu.sync_copy(buf_a, u_out.at[pl.ds(base + h * HBUF, HBUF)])
        pltpu.sync_copy(buf_b, i_out.at[pl.ds(base + h * HBUF, HBUF)])


def _sc_gather(u_tab, i_tab, uidx, iidx):
    mesh = plsc.VectorSubcoreMesh(core_axis_name="c", subcore_axis_name="s")
    run = functools.partial(
        pl.kernel,
        out_type=[jax.ShapeDtypeStruct((B, D2), jnp.float32)] * 2,
        mesh=mesh,
        scratch_types=[
            pltpu.VMEM((NCK, CHUNK), jnp.int32),
            pltpu.VMEM((NCK, CHUNK), jnp.int32),
            pltpu.VMEM((HBUF, D2), jnp.float32),
            pltpu.VMEM((HBUF, D2), jnp.float32),
            pltpu.SemaphoreType.DMA,
            pltpu.SemaphoreType.DMA,
        ],
    )(_sc_gather_body)
    return run(u_tab, i_tab, uidx, iidx)


TILE = 2048


def _dense_body(u_r, i_r, b1, w2, b2, w3, b3, wo2, bo, out):
    gu = u_r[:, :D]
    mu = u_r[:, D:]
    gi = i_r[:, :D]
    mi = i_r[:, D:]
    h = jnp.maximum(mu + mi + b1[...], 0.0)
    h = jnp.maximum(
        jnp.dot(h, w2[...], preferred_element_type=jnp.float32) + b2[...], 0.0)
    h = jnp.maximum(
        jnp.dot(h, w3[...], preferred_element_type=jnp.float32) + b3[...], 0.0)
    logit = (jnp.sum(gu * gi, axis=1, keepdims=True)
             + jnp.sum(h * wo2[...], axis=1, keepdims=True) + bo[...])
    out[...] = 1.0 / (1.0 + jnp.exp(-logit))


def _dense(u_r, i_r, b1, w2, b2, w3, b3, wo2, bo):
    row_spec = pl.BlockSpec((TILE, D2), lambda i: (i, 0))
    full = lambda shape: pl.BlockSpec(shape, lambda i: (0, 0))
    return pl.pallas_call(
        _dense_body,
        grid=(B // TILE,),
        in_specs=[
            row_spec, row_spec,
            full((1, 64)),
            full((64, 32)), full((1, 32)),
            full((32, 16)), full((1, 16)),
            full((1, 16)), full((1, 1)),
        ],
        out_specs=pl.BlockSpec((TILE, 1), lambda i: (i, 0)),
        out_shape=jax.ShapeDtypeStruct((B, 1), jnp.float32),
    )(u_r, i_r, b1, w2, b2, w3, b3, wo2, bo)


def kernel(user_input, item_input, gmf_user, gmf_item, mlp_user, mlp_item,
           W1, b1, W2, b2, W3, b3, Wo, bo):
    uidx = user_input.astype(jnp.int32).reshape(B // CHUNK, CHUNK)
    iidx = item_input.astype(jnp.int32).reshape(B // CHUNK, CHUNK)

    ones = jnp.ones((), jnp.float32)
    diag_wo = jnp.diag(Wo[:D, 0])
    diag_one = jnp.diag(jnp.broadcast_to(ones, (D,)))
    u_tab, i_tab = _stage1(
        gmf_user.T, gmf_item.T, mlp_user.T, mlp_item.T,
        diag_wo, diag_one, W1[:D], W1[D:])

    u_rows, i_rows = _sc_gather(u_tab, i_tab, uidx, iidx)

    return _dense(
        u_rows, i_rows,
        b1.reshape(1, 64), W2, b2.reshape(1, 32), W3, b3.reshape(1, 16),
        Wo[D:, 0].reshape(1, 16), bo.reshape(1, 1))


# speedup vs baseline: 1.1847x; 1.0135x over previous
import functools

import jax
import jax.numpy as jnp
from jax import lax
from jax.experimental import pallas as pl
from jax.experimental.pallas import tpu as pltpu
from jax.experimental.pallas import tpu_sc as plsc

B = 16384
D = 64
D2 = 2 * D
NC = 2
NS = 16
NW = NC * NS
BPW = B // NW
HBUF = 256
CHUNK = 128
NCK = BPW // CHUNK

CB = 8192


def _stage1_body(gu_t, gi_t, mu_t, mi_t, wgu, wgi, wmu, wmi, u_o, i_o):
    dn = (((0,), (0,)), ((), ()))

    def two(a_t, wa, b_t, wb):
        a = lax.dot_general(a_t[...], wa[...], dimension_numbers=dn,
                            preferred_element_type=jnp.float32)
        b = lax.dot_general(b_t[...], wb[...], dimension_numbers=dn,
                            preferred_element_type=jnp.float32)
        full = jnp.concatenate([a, b], axis=1)
        return pltpu.bitcast(full.astype(jnp.bfloat16), jnp.float32)

    u_o[...] = two(gu_t, wgu, mu_t, wmu)
    i_o[...] = two(gi_t, wgi, mi_t, wmi)


def _stage1(gu_t, gi_t, mu_t, mi_t, wgu, wgi, wmu, wmi):
    n = gu_t.shape[1]
    col_spec = pl.BlockSpec((D, CB), lambda i: (0, i))
    w_spec = pl.BlockSpec((D, D), lambda i: (0, 0))
    out_spec = pl.BlockSpec((CB // 2, D2), lambda i: (i, 0))
    return pl.pallas_call(
        _stage1_body,
        grid=(pl.cdiv(n, CB),),
        in_specs=[col_spec] * 4 + [w_spec] * 4,
        out_specs=[out_spec] * 2,
        out_shape=[jax.ShapeDtypeStruct((n // 2, D2), jnp.float32)] * 2,
    )(gu_t, gi_t, mu_t, mi_t, wgu, wgi, wmu, wmi)


def _sc_gather_body(u_tab, i_tab, uidx, iidx, u_out, i_out,
                    uidx_v, iidx_v, buf_a, buf_b, sem_a, sem_b):
    wid = lax.axis_index("s") * NC + lax.axis_index("c")
    base = wid * BPW
    row = wid * NCK
    pltpu.sync_copy(uidx.at[pl.ds(row, NCK)], uidx_v)
    pltpu.sync_copy(iidx.at[pl.ds(row, NCK)], iidx_v)

    for h in range(BPW // HBUF):
        cps = []
        for j in range(HBUF // CHUNK):
            c = h * (HBUF // CHUNK) + j
            cps.append(pltpu.async_copy(
                u_tab.at[uidx_v.at[c]],
                buf_a.at[pl.ds(j * CHUNK, CHUNK)], sem_a))
            cps.append(pltpu.async_copy(
                i_tab.at[iidx_v.at[c]],
                buf_b.at[pl.ds(j * CHUNK, CHUNK)], sem_b))
        for cp in cps:
            cp.wait()
        pltpu.sync_copy(buf_a, u_out.at[pl.ds(base + h * HBUF, HBUF)])
        pltpu.sync_copy(buf_b, i_out.at[pl.ds(base + h * HBUF, HBUF)])


def _sc_gather(u_tab, i_tab, uidx, iidx):
    mesh = plsc.VectorSubcoreMesh(core_axis_name="c", subcore_axis_name="s")
    run = functools.partial(
        pl.kernel,
        out_type=[jax.ShapeDtypeStruct((B, D2), jnp.float32)] * 2,
        mesh=mesh,
        scratch_types=[
            pltpu.VMEM((NCK, CHUNK), jnp.int32),
            pltpu.VMEM((NCK, CHUNK), jnp.int32),
            pltpu.VMEM((HBUF, D2), jnp.float32),
            pltpu.VMEM((HBUF, D2), jnp.float32),
            pltpu.SemaphoreType.DMA,
            pltpu.SemaphoreType.DMA,
        ],
    )(_sc_gather_body)
    return run(u_tab, i_tab, uidx, iidx)


TILE = 2048


def _dense_body(u_r, i_r, pu, pi, b1, w2, b2, w3, b3, wo2, bo, out):
    def unpack(packed, parity):
        w = lax.bitcast_convert_type(packed, jnp.uint32)
        even = lax.bitcast_convert_type(w << 16, jnp.float32)
        odd = lax.bitcast_convert_type(w & jnp.uint32(0xFFFF0000), jnp.float32)
        return jnp.where(parity > 0.5, odd, even)

    u = unpack(u_r[...], pu[...])
    i = unpack(i_r[...], pi[...])
    gu = u[:, :D]
    mu = u[:, D:]
    gi = i[:, :D]
    mi = i[:, D:]
    h = jnp.maximum(mu + mi + b1[...], 0.0)
    h = jnp.maximum(
        jnp.dot(h, w2[...], preferred_element_type=jnp.float32) + b2[...], 0.0)
    h = jnp.maximum(
        jnp.dot(h, w3[...], preferred_element_type=jnp.float32) + b3[...], 0.0)
    logit = (jnp.sum(gu * gi, axis=1, keepdims=True)
             + jnp.sum(h * wo2[...], axis=1, keepdims=True) + bo[...])
    out[...] = 1.0 / (1.0 + jnp.exp(-logit))


def _dense(u_r, i_r, pu, pi, b1, w2, b2, w3, b3, wo2, bo):
    row_spec = pl.BlockSpec((TILE, D2), lambda i: (i, 0))
    par_spec = pl.BlockSpec((TILE, 1), lambda i: (i, 0))
    full = lambda shape: pl.BlockSpec(shape, lambda i: (0, 0))
    return pl.pallas_call(
        _dense_body,
        grid=(B // TILE,),
        in_specs=[
            row_spec, row_spec, par_spec, par_spec,
            full((1, 64)),
            full((64, 32)), full((1, 32)),
            full((32, 16)), full((1, 16)),
            full((1, 16)), full((1, 1)),
        ],
        out_specs=pl.BlockSpec((TILE, 1), lambda i: (i, 0)),
        out_shape=jax.ShapeDtypeStruct((B, 1), jnp.float32),
    )(u_r, i_r, pu, pi, b1, w2, b2, w3, b3, wo2, bo)


def kernel(user_input, item_input, gmf_user, gmf_item, mlp_user, mlp_item,
           W1, b1, W2, b2, W3, b3, Wo, bo):
    ui32 = user_input.astype(jnp.int32)
    ii32 = item_input.astype(jnp.int32)
    uidx = (ui32 >> 1).reshape(B // CHUNK, CHUNK)
    iidx = (ii32 >> 1).reshape(B // CHUNK, CHUNK)
    pu = (ui32 & 1).astype(jnp.float32).reshape(B, 1)
    pi = (ii32 & 1).astype(jnp.float32).reshape(B, 1)

    ones = jnp.ones((), jnp.float32)
    diag_wo = jnp.diag(Wo[:D, 0])
    diag_one = jnp.diag(jnp.broadcast_to(ones, (D,)))
    u_tab, i_tab = _stage1(
        gmf_user.T, gmf_item.T, mlp_user.T, mlp_item.T,
        diag_wo, diag_one, W1[:D], W1[D:])

    u_rows, i_rows = _sc_gather(u_tab, i_tab, uidx, iidx)

    return _dense(
        u_rows, i_rows, pu, pi,
        b1.reshape(1, 64), W2, b2.reshape(1, 32), W3, b3.reshape(1, 16),
        Wo[D:, 0].reshape(1, 16), bo.reshape(1, 1))


# speedup vs baseline: 1.4661x; 1.2375x over previous
import functools

import jax
import jax.numpy as jnp
from jax import lax
from jax.experimental import pallas as pl
from jax.experimental.pallas import tpu as pltpu
from jax.experimental.pallas import tpu_sc as plsc

B = 16384
D = 64
D2 = 2 * D
NC = 2
NS = 16
NW = NC * NS
BPW = B // NW
HBUF = 256
CHUNK = 128
NCK = BPW // CHUNK

CB = 8192


def _stage1_body(gu_t, gi_t, mu_t, mi_t, wu, wi, u_o, i_o):
    dn = (((0,), (0,)), ((), ()))

    def two(a_t, b_t, w):
        x = jnp.concatenate([a_t[...], b_t[...]], axis=0)
        full = lax.dot_general(x, w[...], dimension_numbers=dn,
                               preferred_element_type=jnp.float32)
        return pltpu.bitcast(full.astype(jnp.bfloat16), jnp.float32)

    u_o[...] = two(gu_t, mu_t, wu)
    i_o[...] = two(gi_t, mi_t, wi)


def _stage1(gu_t, gi_t, mu_t, mi_t, wu, wi):
    n = gu_t.shape[1]
    col_spec = pl.BlockSpec((D, CB), lambda i: (0, i))
    w_spec = pl.BlockSpec((D2, D2), lambda i: (0, 0))
    out_spec = pl.BlockSpec((CB // 2, D2), lambda i: (i, 0))
    return pl.pallas_call(
        _stage1_body,
        grid=(pl.cdiv(n, CB),),
        in_specs=[col_spec] * 4 + [w_spec] * 2,
        out_specs=[out_spec] * 2,
        out_shape=[jax.ShapeDtypeStruct((n // 2, D2), jnp.float32)] * 2,
    )(gu_t, gi_t, mu_t, mi_t, wu, wi)


def _sc_gather_body(u_tab, i_tab, uidx, iidx, u_out, i_out,
                    uidx_v, iidx_v, buf_a, buf_b, sem_a, sem_b):
    wid = lax.axis_index("s") * NC + lax.axis_index("c")
    base = wid * BPW
    row = wid * NCK
    pltpu.sync_copy(uidx.at[pl.ds(row, NCK)], uidx_v)
    pltpu.sync_copy(iidx.at[pl.ds(row, NCK)], iidx_v)

    for h in range(BPW // HBUF):
        cps = []
        for j in range(HBUF // CHUNK):
            c = h * (HBUF // CHUNK) + j
            cps.append(pltpu.async_copy(
                u_tab.at[uidx_v.at[c]],
                buf_a.at[pl.ds(j * CHUNK, CHUNK)], sem_a))
            cps.append(pltpu.async_copy(
                i_tab.at[iidx_v.at[c]],
                buf_b.at[pl.ds(j * CHUNK, CHUNK)], sem_b))
        for cp in cps:
            cp.wait()
        pltpu.sync_copy(buf_a, u_out.at[pl.ds(base + h * HBUF, HBUF)])
        pltpu.sync_copy(buf_b, i_out.at[pl.ds(base + h * HBUF, HBUF)])


def _sc_gather(u_tab, i_tab, uidx, iidx):
    mesh = plsc.VectorSubcoreMesh(core_axis_name="c", subcore_axis_name="s")
    run = functools.partial(
        pl.kernel,
        out_type=[jax.ShapeDtypeStruct((B, D2), jnp.float32)] * 2,
        mesh=mesh,
        scratch_types=[
            pltpu.VMEM((NCK, CHUNK), jnp.int32),
            pltpu.VMEM((NCK, CHUNK), jnp.int32),
            pltpu.VMEM((HBUF, D2), jnp.float32),
            pltpu.VMEM((HBUF, D2), jnp.float32),
            pltpu.SemaphoreType.DMA,
            pltpu.SemaphoreType.DMA,
        ],
    )(_sc_gather_body)
    return run(u_tab, i_tab, uidx, iidx)


TILE = 2048


def _dense_body(u_r, i_r, pu, pi, b1, w2, b2, w3, b3, wo2, bo, out):
    def unpack(packed, parity):
        w = lax.bitcast_convert_type(packed, jnp.uint32)
        even = lax.bitcast_convert_type(w << 16, jnp.float32)
        odd = lax.bitcast_convert_type(w & jnp.uint32(0xFFFF0000), jnp.float32)
        return jnp.where(parity > 0.5, odd, even)

    u = unpack(u_r[...], pu[...])
    i = unpack(i_r[...], pi[...])
    gu = u[:, :D]
    mu = u[:, D:]
    gi = i[:, :D]
    mi = i[:, D:]
    h = jnp.maximum(mu + mi + b1[...], 0.0)
    h = jnp.maximum(
        jnp.dot(h, w2[...], preferred_element_type=jnp.float32) + b2[...], 0.0)
    h = jnp.maximum(
        jnp.dot(h, w3[...], preferred_element_type=jnp.float32) + b3[...], 0.0)
    logit = (jnp.sum(gu * gi, axis=1, keepdims=True)
             + jnp.sum(h * wo2[...], axis=1, keepdims=True) + bo[...])
    out[...] = 1.0 / (1.0 + jnp.exp(-logit))


def _dense(u_r, i_r, pu, pi, b1, w2, b2, w3, b3, wo2, bo):
    row_spec = pl.BlockSpec((TILE, D2), lambda i: (i, 0))
    par_spec = pl.BlockSpec((TILE, 1), lambda i: (i, 0))
    full = lambda shape: pl.BlockSpec(shape, lambda i: (0, 0))
    return pl.pallas_call(
        _dense_body,
        grid=(B // TILE,),
        in_specs=[
            row_spec, row_spec, par_spec, par_spec,
            full((1, 64)),
            full((64, 32)), full((1, 32)),
            full((32, 16)), full((1, 16)),
            full((1, 16)), full((1, 1)),
        ],
        out_specs=pl.BlockSpec((TILE, 1), lambda i: (i, 0)),
        out_shape=jax.ShapeDtypeStruct((B, 1), jnp.float32),
    )(u_r, i_r, pu, pi, b1, w2, b2, w3, b3, wo2, bo)


def kernel(user_input, item_input, gmf_user, gmf_item, mlp_user, mlp_item,
           W1, b1, W2, b2, W3, b3, Wo, bo):
    ui32 = user_input.astype(jnp.int32)
    ii32 = item_input.astype(jnp.int32)
    uidx = (ui32 >> 1).reshape(B // CHUNK, CHUNK)
    iidx = (ii32 >> 1).reshape(B // CHUNK, CHUNK)
    pu = (ui32 & 1).astype(jnp.float32).reshape(B, 1)
    pi = (ii32 & 1).astype(jnp.float32).reshape(B, 1)

    ones = jnp.ones((), jnp.float32)
    diag_wo = jnp.diag(Wo[:D, 0])
    diag_one = jnp.diag(jnp.broadcast_to(ones, (D,)))
    z = jnp.zeros((D, D), jnp.float32)
    wu = jnp.block([[diag_wo, z], [z, W1[:D]]])
    wi = jnp.block([[diag_one, z], [z, W1[D:]]])
    u_tab, i_tab = _stage1(
        gmf_user.T, gmf_item.T, mlp_user.T, mlp_item.T, wu, wi)

    u_rows, i_rows = _sc_gather(u_tab, i_tab, uidx, iidx)

    return _dense(
        u_rows, i_rows, pu, pi,
        b1.reshape(1, 64), W2, b2.reshape(1, 32), W3, b3.reshape(1, 16),
        Wo[D:, 0].reshape(1, 16), bo.reshape(1, 1))


# speedup vs baseline: 1.4863x; 1.0138x over previous
import functools

import jax
import jax.numpy as jnp
from jax import lax
from jax.experimental import pallas as pl
from jax.experimental.pallas import tpu as pltpu
from jax.experimental.pallas import tpu_sc as plsc

B = 16384
D = 64
D2 = 2 * D
NC = 2
NS = 16
NW = NC * NS
BPW = B // NW
HBUF = 256
CHUNK = 128
NCK = BPW // CHUNK

CB = 16384


def _stage1_body(gu_t, gi_t, mu_t, mi_t, wu, wi, u_o, i_o):
    dn = (((0,), (0,)), ((), ()))

    def two(a_t, b_t, w):
        x = jnp.concatenate([a_t[...], b_t[...]], axis=0)
        full = lax.dot_general(x, w[...], dimension_numbers=dn,
                               preferred_element_type=jnp.float32)
        return pltpu.bitcast(full.astype(jnp.bfloat16), jnp.float32)

    u_o[...] = two(gu_t, mu_t, wu)
    i_o[...] = two(gi_t, mi_t, wi)


def _stage1(gu_t, gi_t, mu_t, mi_t, wu, wi):
    n = gu_t.shape[1]
    col_spec = pl.BlockSpec((D, CB), lambda i: (0, i))
    w_spec = pl.BlockSpec((D2, D2), lambda i: (0, 0))
    out_spec = pl.BlockSpec((CB // 2, D2), lambda i: (i, 0))
    return pl.pallas_call(
        _stage1_body,
        grid=(pl.cdiv(n, CB),),
        in_specs=[col_spec] * 4 + [w_spec] * 2,
        out_specs=[out_spec] * 2,
        out_shape=[jax.ShapeDtypeStruct((n // 2, D2), jnp.float32)] * 2,
    )(gu_t, gi_t, mu_t, mi_t, wu, wi)


def _sc_gather_body(u_tab, i_tab, uidx, iidx, u_out, i_out,
                    uidx_v, iidx_v, buf_a, buf_b, sem_a, sem_b):
    wid = lax.axis_index("s") * NC + lax.axis_index("c")
    base = wid * BPW
    row = wid * NCK
    pltpu.sync_copy(uidx.at[pl.ds(row, NCK)], uidx_v)
    pltpu.sync_copy(iidx.at[pl.ds(row, NCK)], iidx_v)

    for h in range(BPW // HBUF):
        cps = []
        for j in range(HBUF // CHUNK):
            c = h * (HBUF // CHUNK) + j
            cps.append(pltpu.async_copy(
                u_tab.at[uidx_v.at[c]],
                buf_a.at[pl.ds(j * CHUNK, CHUNK)], sem_a))
            cps.append(pltpu.async_copy(
                i_tab.at[iidx_v.at[c]],
                buf_b.at[pl.ds(j * CHUNK, CHUNK)], sem_b))
        for cp in cps:
            cp.wait()
        pltpu.sync_copy(buf_a, u_out.at[pl.ds(base + h * HBUF, HBUF)])
        pltpu.sync_copy(buf_b, i_out.at[pl.ds(base + h * HBUF, HBUF)])


def _sc_gather(u_tab, i_tab, uidx, iidx):
    mesh = plsc.VectorSubcoreMesh(core_axis_name="c", subcore_axis_name="s")
    run = functools.partial(
        pl.kernel,
        out_type=[jax.ShapeDtypeStruct((B, D2), jnp.float32)] * 2,
        mesh=mesh,
        scratch_types=[
            pltpu.VMEM((NCK, CHUNK), jnp.int32),
            pltpu.VMEM((NCK, CHUNK), jnp.int32),
            pltpu.VMEM((HBUF, D2), jnp.float32),
            pltpu.VMEM((HBUF, D2), jnp.float32),
            pltpu.SemaphoreType.DMA,
            pltpu.SemaphoreType.DMA,
        ],
    )(_sc_gather_body)
    return run(u_tab, i_tab, uidx, iidx)


TILE = 2048


def _dense_body(u_r, i_r, pu, pi, b1, w2, b2, w3, b3, wo2, bo, out):
    def unpack(packed, parity):
        w = lax.bitcast_convert_type(packed, jnp.uint32)
        even = lax.bitcast_convert_type(w << 16, jnp.float32)
        odd = lax.bitcast_convert_type(w & jnp.uint32(0xFFFF0000), jnp.float32)
        return jnp.where(parity > 0.5, odd, even)

    u = unpack(u_r[...], pu[...])
    i = unpack(i_r[...], pi[...])
    gu = u[:, :D]
    mu = u[:, D:]
    gi = i[:, :D]
    mi = i[:, D:]
    h = jnp.maximum(mu + mi + b1[...], 0.0)
    h = jnp.maximum(
        jnp.dot(h, w2[...], preferred_element_type=jnp.float32) + b2[...], 0.0)
    h = jnp.maximum(
        jnp.dot(h, w3[...], preferred_element_type=jnp.float32) + b3[...], 0.0)
    logit = (jnp.sum(gu * gi, axis=1, keepdims=True)
             + jnp.sum(h * wo2[...], axis=1, keepdims=True) + bo[...])
    out[...] = 1.0 / (1.0 + jnp.exp(-logit))


def _dense(u_r, i_r, pu, pi, b1, w2, b2, w3, b3, wo2, bo):
    row_spec = pl.BlockSpec((TILE, D2), lambda i: (i, 0))
    par_spec = pl.BlockSpec((TILE, 1), lambda i: (i, 0))
    full = lambda shape: pl.BlockSpec(shape, lambda i: (0, 0))
    return pl.pallas_call(
        _dense_body,
        grid=(B // TILE,),
        in_specs=[
            row_spec, row_spec, par_spec, par_spec,
            full((1, 64)),
            full((64, 32)), full((1, 32)),
            full((32, 16)), full((1, 16)),
            full((1, 16)), full((1, 1)),
        ],
        out_specs=pl.BlockSpec((TILE, 1), lambda i: (i, 0)),
        out_shape=jax.ShapeDtypeStruct((B, 1), jnp.float32),
    )(u_r, i_r, pu, pi, b1, w2, b2, w3, b3, wo2, bo)


def kernel(user_input, item_input, gmf_user, gmf_item, mlp_user, mlp_item,
           W1, b1, W2, b2, W3, b3, Wo, bo):
    ui32 = user_input.astype(jnp.int32)
    ii32 = item_input.astype(jnp.int32)
    uidx = (ui32 >> 1).reshape(B // CHUNK, CHUNK)
    iidx = (ii32 >> 1).reshape(B // CHUNK, CHUNK)
    pu = (ui32 & 1).astype(jnp.float32).reshape(B, 1)
    pi = (ii32 & 1).astype(jnp.float32).reshape(B, 1)

    ones = jnp.ones((), jnp.float32)
    diag_wo = jnp.diag(Wo[:D, 0])
    diag_one = jnp.diag(jnp.broadcast_to(ones, (D,)))
    z = jnp.zeros((D, D), jnp.float32)
    wu = jnp.block([[diag_wo, z], [z, W1[:D]]])
    wi = jnp.block([[diag_one, z], [z, W1[D:]]])
    u_tab, i_tab = _stage1(
        gmf_user.T, gmf_item.T, mlp_user.T, mlp_item.T, wu, wi)

    u_rows, i_rows = _sc_gather(u_tab, i_tab, uidx, iidx)

    return _dense(
        u_rows, i_rows, pu, pi,
        b1.reshape(1, 64), W2, b2.reshape(1, 32), W3, b3.reshape(1, 16),
        Wo[D:, 0].reshape(1, 16), bo.reshape(1, 1))


# speedup vs baseline: 1.4994x; 1.0088x over previous
import functools

import jax
import jax.numpy as jnp
from jax import lax
from jax.experimental import pallas as pl
from jax.experimental.pallas import tpu as pltpu
from jax.experimental.pallas import tpu_sc as plsc

B = 16384
D = 64
D2 = 2 * D
NC = 2
NS = 16
NW = NC * NS
BPW = B // NW
HBUF = 256
CHUNK = 128
NCK = BPW // CHUNK

CB = 16384


def _stage1_body(gu_t, gi_t, mu_t, mi_t, wu, wi, u_o, i_o):
    dn = (((0,), (0,)), ((), ()))

    def two(a_t, b_t, w):
        x = jnp.concatenate([a_t[...], b_t[...]], axis=0)
        full = lax.dot_general(x.astype(jnp.bfloat16), w[...],
                               dimension_numbers=dn,
                               preferred_element_type=jnp.float32)
        return pltpu.bitcast(full.astype(jnp.bfloat16), jnp.float32)

    u_o[...] = two(gu_t, mu_t, wu)
    i_o[...] = two(gi_t, mi_t, wi)


def _stage1(gu_t, gi_t, mu_t, mi_t, wu, wi):
    n = gu_t.shape[1]
    col_spec = pl.BlockSpec((D, CB), lambda i: (0, i))
    w_spec = pl.BlockSpec((D2, D2), lambda i: (0, 0))
    out_spec = pl.BlockSpec((CB // 2, D2), lambda i: (i, 0))
    return pl.pallas_call(
        _stage1_body,
        grid=(pl.cdiv(n, CB),),
        in_specs=[col_spec] * 4 + [w_spec] * 2,
        out_specs=[out_spec] * 2,
        out_shape=[jax.ShapeDtypeStruct((n // 2, D2), jnp.float32)] * 2,
    )(gu_t, gi_t, mu_t, mi_t, wu, wi)


def _sc_gather_body(u_tab, i_tab, uidx, iidx, u_out, i_out,
                    uidx_v, iidx_v, buf_a, buf_b, sem_a, sem_b):
    wid = lax.axis_index("s") * NC + lax.axis_index("c")
    base = wid * BPW
    row = wid * NCK
    pltpu.sync_copy(uidx.at[pl.ds(row, NCK)], uidx_v)
    pltpu.sync_copy(iidx.at[pl.ds(row, NCK)], iidx_v)

    for h in range(BPW // HBUF):
        cps = []
        for j in range(HBUF // CHUNK):
            c = h * (HBUF // CHUNK) + j
            cps.append(pltpu.async_copy(
                u_tab.at[uidx_v.at[c]],
                buf_a.at[pl.ds(j * CHUNK, CHUNK)], sem_a))
            cps.append(pltpu.async_copy(
                i_tab.at[iidx_v.at[c]],
                buf_b.at[pl.ds(j * CHUNK, CHUNK)], sem_b))
        for cp in cps:
            cp.wait()
        pltpu.sync_copy(buf_a, u_out.at[pl.ds(base + h * HBUF, HBUF)])
        pltpu.sync_copy(buf_b, i_out.at[pl.ds(base + h * HBUF, HBUF)])


def _sc_gather(u_tab, i_tab, uidx, iidx):
    mesh = plsc.VectorSubcoreMesh(core_axis_name="c", subcore_axis_name="s")
    run = functools.partial(
        pl.kernel,
        out_type=[jax.ShapeDtypeStruct((B, D2), jnp.float32)] * 2,
        mesh=mesh,
        scratch_types=[
            pltpu.VMEM((NCK, CHUNK), jnp.int32),
            pltpu.VMEM((NCK, CHUNK), jnp.int32),
            pltpu.VMEM((HBUF, D2), jnp.float32),
            pltpu.VMEM((HBUF, D2), jnp.float32),
            pltpu.SemaphoreType.DMA,
            pltpu.SemaphoreType.DMA,
        ],
    )(_sc_gather_body)
    return run(u_tab, i_tab, uidx, iidx)


TILE = 2048


def _dense_body(u_r, i_r, pu, pi, b1, w2, b2, w3, b3, wo2, bo, out):
    def unpack(packed, parity):
        w = lax.bitcast_convert_type(packed, jnp.uint32)
        even = lax.bitcast_convert_type(w << 16, jnp.float32)
        odd = lax.bitcast_convert_type(w & jnp.uint32(0xFFFF0000), jnp.float32)
        return jnp.where(parity > 0.5, odd, even)

    u = unpack(u_r[...], pu[...])
    i = unpack(i_r[...], pi[...])
    gu = u[:, :D]
    mu = u[:, D:]
    gi = i[:, :D]
    mi = i[:, D:]
    h = jnp.maximum(mu + mi + b1[...], 0.0)
    h = jnp.maximum(
        jnp.dot(h, w2[...], preferred_element_type=jnp.float32) + b2[...], 0.0)
    h = jnp.maximum(
        jnp.dot(h, w3[...], preferred_element_type=jnp.float32) + b3[...], 0.0)
    logit = (jnp.sum(gu * gi, axis=1, keepdims=True)
             + jnp.sum(h * wo2[...], axis=1, keepdims=True) + bo[...])
    out[...] = 1.0 / (1.0 + jnp.exp(-logit))


def _dense(u_r, i_r, pu, pi, b1, w2, b2, w3, b3, wo2, bo):
    row_spec = pl.BlockSpec((TILE, D2), lambda i: (i, 0))
    par_spec = pl.BlockSpec((TILE, 1), lambda i: (i, 0))
    full = lambda shape: pl.BlockSpec(shape, lambda i: (0, 0))
    return pl.pallas_call(
        _dense_body,
        grid=(B // TILE,),
        in_specs=[
            row_spec, row_spec, par_spec, par_spec,
            full((1, 64)),
            full((64, 32)), full((1, 32)),
            full((32, 16)), full((1, 16)),
            full((1, 16)), full((1, 1)),
        ],
        out_specs=pl.BlockSpec((TILE, 1), lambda i: (i, 0)),
        out_shape=jax.ShapeDtypeStruct((B, 1), jnp.float32),
    )(u_r, i_r, pu, pi, b1, w2, b2, w3, b3, wo2, bo)


def kernel(user_input, item_input, gmf_user, gmf_item, mlp_user, mlp_item,
           W1, b1, W2, b2, W3, b3, Wo, bo):
    ui32 = user_input.astype(jnp.int32)
    ii32 = item_input.astype(jnp.int32)
    uidx = (ui32 >> 1).reshape(B // CHUNK, CHUNK)
    iidx = (ii32 >> 1).reshape(B // CHUNK, CHUNK)
    pu = (ui32 & 1).astype(jnp.float32).reshape(B, 1)
    pi = (ii32 & 1).astype(jnp.float32).reshape(B, 1)

    ones = jnp.ones((), jnp.float32)
    diag_wo = jnp.diag(Wo[:D, 0])
    diag_one = jnp.diag(jnp.broadcast_to(ones, (D,)))
    z = jnp.zeros((D, D), jnp.float32)
    wu = jnp.block([[diag_wo, z], [z, W1[:D]]])
    wi = jnp.block([[diag_one, z], [z, W1[D:]]])
    u_tab, i_tab = _stage1(
        gmf_user.T, gmf_item.T, mlp_user.T, mlp_item.T,
        wu.astype(jnp.bfloat16), wi.astype(jnp.bfloat16))

    u_rows, i_rows = _sc_gather(u_tab, i_tab, uidx, iidx)

    return _dense(
        u_rows, i_rows, pu, pi,
        b1.reshape(1, 64), W2, b2.reshape(1, 32), W3, b3.reshape(1, 16),
        Wo[D:, 0].reshape(1, 16), bo.reshape(1, 1))


# speedup vs baseline: 2.0108x; 1.3411x over previous
import functools

import jax
import jax.numpy as jnp
from jax import lax
from jax.experimental import pallas as pl
from jax.experimental.pallas import tpu as pltpu
from jax.experimental.pallas import tpu_sc as plsc

B = 16384
D = 64
D2 = 2 * D
NC = 2
NS = 16
NW = NC * NS
BPW = B // NW
HBUF = 256
CHUNK = 128
NCK = BPW // CHUNK

CB = 16384


def _stage1_body(a_t, b_t, w, o):
    dn = (((0,), (0,)), ((), ()))
    x = jnp.concatenate([a_t[...], b_t[...]], axis=0)
    full = lax.dot_general(x.astype(jnp.bfloat16), w[...],
                           dimension_numbers=dn,
                           preferred_element_type=jnp.float32)
    o[...] = pltpu.bitcast(full.astype(jnp.bfloat16), jnp.float32)


def _stage1(a_t, b_t, w):
    n = a_t.shape[1]
    col_spec = pl.BlockSpec((D, CB), lambda i: (0, i))
    w_spec = pl.BlockSpec((D2, D2), lambda i: (0, 0))
    out_spec = pl.BlockSpec((CB // 2, D2), lambda i: (i, 0))
    return pl.pallas_call(
        _stage1_body,
        grid=(pl.cdiv(n, CB),),
        in_specs=[col_spec] * 2 + [w_spec],
        out_specs=out_spec,
        out_shape=jax.ShapeDtypeStruct((n // 2, D2), jnp.float32),
    )(a_t, b_t, w)


def _sc_gather_body(tab, idx, out,
                    idx_v, buf_a, buf_b, sem_a, sem_b):
    wid = lax.axis_index("s") * NC + lax.axis_index("c")
    base = wid * BPW
    row = wid * NCK
    pltpu.sync_copy(idx.at[pl.ds(row, NCK)], idx_v)

    for h in range(BPW // HBUF):
        buf = buf_a if h % 2 == 0 else buf_b
        sem = sem_a if h % 2 == 0 else sem_b
        cps = []
        for j in range(HBUF // CHUNK):
            c = h * (HBUF // CHUNK) + j
            cps.append(pltpu.async_copy(
                tab.at[idx_v.at[c]],
                buf.at[pl.ds(j * CHUNK, CHUNK)], sem))
        for cp in cps:
            cp.wait()
        pltpu.sync_copy(buf, out.at[pl.ds(base + h * HBUF, HBUF)])


def _sc_gather(tab, idx):
    mesh = plsc.VectorSubcoreMesh(core_axis_name="c", subcore_axis_name="s")
    run = functools.partial(
        pl.kernel,
        out_type=jax.ShapeDtypeStruct((B, D2), jnp.float32),
        mesh=mesh,
        scratch_types=[
            pltpu.VMEM((NCK, CHUNK), jnp.int32),
            pltpu.VMEM((HBUF, D2), jnp.float32),
            pltpu.VMEM((HBUF, D2), jnp.float32),
            pltpu.SemaphoreType.DMA,
            pltpu.SemaphoreType.DMA,
        ],
    )(_sc_gather_body)
    return run(tab, idx)


TILE = 2048


def _dense_body(u_r, i_r, pu, pi, b1, w2, b2, w3, b3, wo2, bo, out):
    def unpack(packed, parity):
        w = lax.bitcast_convert_type(packed, jnp.uint32)
        even = lax.bitcast_convert_type(w << 16, jnp.float32)
        odd = lax.bitcast_convert_type(w & jnp.uint32(0xFFFF0000), jnp.float32)
        return jnp.where(parity > 0.5, odd, even)

    u = unpack(u_r[...], pu[...])
    i = unpack(i_r[...], pi[...])
    gu = u[:, :D]
    mu = u[:, D:]
    gi = i[:, :D]
    mi = i[:, D:]
    h = jnp.maximum(mu + mi + b1[...], 0.0)
    h = jnp.maximum(
        jnp.dot(h, w2[...], preferred_element_type=jnp.float32) + b2[...], 0.0)
    h = jnp.maximum(
        jnp.dot(h, w3[...], preferred_element_type=jnp.float32) + b3[...], 0.0)
    logit = (jnp.sum(gu * gi, axis=1, keepdims=True)
             + jnp.sum(h * wo2[...], axis=1, keepdims=True) + bo[...])
    out[...] = 1.0 / (1.0 + jnp.exp(-logit))


def _dense(u_r, i_r, pu, pi, b1, w2, b2, w3, b3, wo2, bo):
    row_spec = pl.BlockSpec((TILE, D2), lambda i: (i, 0))
    par_spec = pl.BlockSpec((TILE, 1), lambda i: (i, 0))
    full = lambda shape: pl.BlockSpec(shape, lambda i: (0, 0))
    return pl.pallas_call(
        _dense_body,
        grid=(B // TILE,),
        in_specs=[
            row_spec, row_spec, par_spec, par_spec,
            full((1, 64)),
            full((64, 32)), full((1, 32)),
            full((32, 16)), full((1, 16)),
            full((1, 16)), full((1, 1)),
        ],
        out_specs=pl.BlockSpec((TILE, 1), lambda i: (i, 0)),
        out_shape=jax.ShapeDtypeStruct((B, 1), jnp.float32),
    )(u_r, i_r, pu, pi, b1, w2, b2, w3, b3, wo2, bo)


def kernel(user_input, item_input, gmf_user, gmf_item, mlp_user, mlp_item,
           W1, b1, W2, b2, W3, b3, Wo, bo):
    ui32 = user_input.astype(jnp.int32)
    ii32 = item_input.astype(jnp.int32)
    uidx = (ui32 >> 1).reshape(B // CHUNK, CHUNK)
    iidx = (ii32 >> 1).reshape(B // CHUNK, CHUNK)
    pu = (ui32 & 1).astype(jnp.float32).reshape(B, 1)
    pi = (ii32 & 1).astype(jnp.float32).reshape(B, 1)

    ones = jnp.ones((), jnp.float32)
    diag_wo = jnp.diag(Wo[:D, 0])
    diag_one = jnp.diag(jnp.broadcast_to(ones, (D,)))
    z = jnp.zeros((D, D), jnp.float32)
    wu = jnp.block([[diag_wo, z], [z, W1[:D]]])
    wi = jnp.block([[diag_one, z], [z, W1[D:]]])
    u_tab = _stage1(gmf_user.T, mlp_user.T, wu.astype(jnp.bfloat16))
    u_rows = _sc_gather(u_tab, uidx)
    i_tab = _stage1(gmf_item.T, mlp_item.T, wi.astype(jnp.bfloat16))
    i_rows = _sc_gather(i_tab, iidx)

    return _dense(
        u_rows, i_rows, pu, pi,
        b1.reshape(1, 64), W2, b2.reshape(1, 32), W3, b3.reshape(1, 16),
        Wo[D:, 0].reshape(1, 16), bo.reshape(1, 1))
